# baseline (speedup 1.0000x reference)
import functools

import jax
import jax.numpy as jnp
from jax import lax
from jax.experimental import pallas as pl
from jax.experimental.pallas import tpu as pltpu
from jax.experimental.pallas import tpu_sc as plsc

_B, _H, _W, _D = 8, 32, 32, 32
_K = 8192
_M = _B * _H * _W
_MB = 1024
_G = _M // _MB


_KCH = 2048


def _dist_argmin_body(x_ref, e_ref, idx_ref, et_ref):
    i = pl.program_id(0)
    f = x_ref[...]
    e = e_ref[...]
    fsq = jnp.sum(f * f, axis=1, keepdims=True)
    dist = fsq - jnp.dot(
        (2.0 * f).astype(jnp.bfloat16),
        e.astype(jnp.bfloat16),
        preferred_element_type=jnp.float32,
    )
    dist = dist + jnp.sum(e * e, axis=0, keepdims=True)
    acc_v = jnp.full((_MB,), jnp.inf, dtype=jnp.float32)
    acc_i = jnp.zeros((_MB,), dtype=jnp.int32)
    for c in range(_K // _KCH):
        dch = dist[:, c * _KCH:(c + 1) * _KCH]
        mv = jnp.min(dch, axis=1)
        ai = jnp.argmin(dch, axis=1).astype(jnp.int32)
        take = mv < acc_v
        acc_i = jnp.where(take, ai + c * _KCH, acc_i)
        acc_v = jnp.where(take, mv.astype(jnp.bfloat16).astype(jnp.float32), acc_v)
    idx_ref[0, 0, :] = acc_i
    et_ref[...] = e_ref[:, pl.ds(i * _MB, _MB)].T


_dist_argmin = pl.pallas_call(
    _dist_argmin_body,
    grid=(_G,),
    in_specs=[
        pl.BlockSpec((_MB, _D), lambda i: (i, 0)),
        pl.BlockSpec((_D, _K), lambda i: (0, 0)),
    ],
    out_specs=[
        pl.BlockSpec((1, 1, _MB), lambda i: (i, 0, 0)),
        pl.BlockSpec((_MB, _D), lambda i: (i, 0)),
    ],
    out_shape=[
        jax.ShapeDtypeStruct((_G, 1, _MB), jnp.int32),
        jax.ShapeDtypeStruct((_K, _D), jnp.float32),
    ],
    compiler_params=pltpu.CompilerParams(dimension_semantics=("parallel",)),
)


_SC_INFO = plsc.get_sparse_core_info()
_NC, _NS = _SC_INFO.num_cores, _SC_INFO.num_subcores
_NW = _NC * _NS
_BPW = _M // _NW
_CH = 128
_NCH = _BPW // _CH


@functools.partial(
    pl.kernel,
    mesh=plsc.VectorSubcoreMesh(core_axis_name="c", subcore_axis_name="s"),
    out_type=jax.ShapeDtypeStruct((_M, _D), jnp.float32),
    scratch_types=[
        pltpu.VMEM((_NCH, _CH), jnp.int32),
        pltpu.VMEM((_BPW, _D), jnp.float32),
        pltpu.SemaphoreType.DMA,
    ],
    compiler_params=pltpu.CompilerParams(use_tc_tiling_on_sc=False),
)
def _sc_gather(table_hbm, idx_hbm, out_hbm, idx_v, rows_v, sem):
    wid = lax.axis_index("s") * _NC + lax.axis_index("c")
    pltpu.sync_copy(idx_hbm.at[wid], idx_v)
    for c in range(_NCH):
        pltpu.async_copy(
            table_hbm.at[idx_v.at[c]], rows_v.at[pl.ds(c * _CH, _CH)], sem
        ).wait()
    pltpu.sync_copy(rows_v, out_hbm.at[pl.ds(wid * _BPW, _BPW)])


def kernel(x, embed):
    x_flat = x.reshape(_M, _D)
    idx, table = _dist_argmin(x_flat, embed)
    idx = idx.reshape(_NW, _NCH, _CH)
    q = _sc_gather(table, idx)
    return q.reshape(x.shape)


# speedup vs baseline: 1.2473x; 1.2473x over previous
import functools

import jax
import jax.numpy as jnp
from jax import lax
from jax.experimental import pallas as pl
from jax.experimental.pallas import tpu as pltpu
from jax.experimental.pallas import tpu_sc as plsc

_B, _H, _W, _D = 8, 32, 32, 32
_K = 8192
_M = _B * _H * _W
_MB = 1024
_G = _M // _MB


_KCH = 2048


def _dist_argmin_body(x_ref, e_ref, idx_ref, et_ref):
    i = pl.program_id(0)
    f = x_ref[...]
    e = e_ref[...]
    fsq = jnp.sum(f * f, axis=1, keepdims=True)
    mm = jnp.dot(
        (2.0 * f).astype(jnp.bfloat16),
        e.astype(jnp.bfloat16),
        preferred_element_type=jnp.float32,
    )
    esq = jnp.sum(e * e, axis=0, keepdims=True)
    lane = lax.broadcasted_iota(jnp.int32, (_MB, 128), 1).astype(jnp.float32)
    acc_v = jnp.full((_MB,), jnp.inf, dtype=jnp.float32)
    acc_i = jnp.zeros((_MB,), dtype=jnp.float32)
    for c in range(_K // _KCH):
        val = jnp.full((_MB, 128), jnp.inf, dtype=jnp.float32)
        tdx = jnp.zeros((_MB, 128), dtype=jnp.float32)
        for t in range(_KCH // 128):
            s = c * _KCH + t * 128
            d_t = (fsq - mm[:, s:s + 128]) + esq[:, s:s + 128]
            lt = d_t < val
            val = jnp.where(lt, d_t, val)
            tdx = jnp.where(lt, jnp.float32(t), tdx)
        mv = jnp.min(val, axis=1)
        key = tdx * 128.0 + lane
        ai = jnp.min(jnp.where(val == mv[:, None], key, jnp.float32(_K)), axis=1)
        take = mv < acc_v
        acc_i = jnp.where(take, ai + c * _KCH, acc_i)
        acc_v = jnp.where(take, mv.astype(jnp.bfloat16).astype(jnp.float32), acc_v)
    idx_ref[0, 0, :] = acc_i.astype(jnp.int32)
    et_ref[...] = e_ref[:, pl.ds(i * _MB, _MB)].T


_dist_argmin = pl.pallas_call(
    _dist_argmin_body,
    grid=(_G,),
    in_specs=[
        pl.BlockSpec((_MB, _D), lambda i: (i, 0)),
        pl.BlockSpec((_D, _K), lambda i: (0, 0)),
    ],
    out_specs=[
        pl.BlockSpec((1, 1, _MB), lambda i: (i, 0, 0)),
        pl.BlockSpec((_MB, _D), lambda i: (i, 0)),
    ],
    out_shape=[
        jax.ShapeDtypeStruct((_G, 1, _MB), jnp.int32),
        jax.ShapeDtypeStruct((_K, _D), jnp.float32),
    ],
    compiler_params=pltpu.CompilerParams(dimension_semantics=("parallel",)),
)


_SC_INFO = plsc.get_sparse_core_info()
_NC, _NS = _SC_INFO.num_cores, _SC_INFO.num_subcores
_NW = _NC * _NS
_BPW = _M // _NW
_CH = 128
_NCH = _BPW // _CH


@functools.partial(
    pl.kernel,
    mesh=plsc.VectorSubcoreMesh(core_axis_name="c", subcore_axis_name="s"),
    out_type=jax.ShapeDtypeStruct((_M, _D), jnp.float32),
    scratch_types=[
        pltpu.VMEM((_NCH, _CH), jnp.int32),
        pltpu.VMEM((_BPW, _D), jnp.float32),
        pltpu.SemaphoreType.DMA,
    ],
    compiler_params=pltpu.CompilerParams(use_tc_tiling_on_sc=False),
)
def _sc_gather(table_hbm, idx_hbm, out_hbm, idx_v, rows_v, sem):
    wid = lax.axis_index("s") * _NC + lax.axis_index("c")
    pltpu.sync_copy(idx_hbm.at[wid], idx_v)
    for c in range(_NCH):
        pltpu.async_copy(
            table_hbm.at[idx_v.at[c]], rows_v.at[pl.ds(c * _CH, _CH)], sem
        ).wait()
    pltpu.sync_copy(rows_v, out_hbm.at[pl.ds(wid * _BPW, _BPW)])


def kernel(x, embed):
    x_flat = x.reshape(_M, _D)
    idx, table = _dist_argmin(x_flat, embed)
    idx = idx.reshape(_NW, _NCH, _CH)
    q = _sc_gather(table, idx)
    return q.reshape(x.shape)


# speedup vs baseline: 1.3517x; 1.0837x over previous
import functools

import jax
import jax.numpy as jnp
from jax import lax
from jax.experimental import pallas as pl
from jax.experimental.pallas import tpu as pltpu
from jax.experimental.pallas import tpu_sc as plsc

_B, _H, _W, _D = 8, 32, 32, 32
_K = 8192
_M = _B * _H * _W
_MB = 1024
_G = _M // _MB
_KCH = 2048

_SC_INFO = plsc.get_sparse_core_info()
_NC, _NS = _SC_INFO.num_cores, _SC_INFO.num_subcores
_NW = _NC * _NS
_BPW = _M // _NW
_CH = 128
_NCH = _BPW // _CH
_WPG = _NW // _G


def _dist_argmin_body(x_ref, e_ref, idx_ref, et_ref):
    i = pl.program_id(0)
    f = x_ref[...].reshape(_MB, _D)
    e = e_ref[...]
    fsq = jnp.sum(f * f, axis=1, keepdims=True)
    mm = jnp.dot(
        (2.0 * f).astype(jnp.bfloat16),
        e.astype(jnp.bfloat16),
        preferred_element_type=jnp.float32,
    )
    esq = jnp.sum(e * e, axis=0, keepdims=True)
    lane = lax.broadcasted_iota(jnp.int32, (_MB, 128), 1).astype(jnp.float32)
    acc_v = jnp.full((_MB,), jnp.inf, dtype=jnp.float32)
    acc_i = jnp.zeros((_MB,), dtype=jnp.float32)
    for c in range(_K // _KCH):
        val = jnp.full((_MB, 128), jnp.inf, dtype=jnp.float32)
        tdx = jnp.zeros((_MB, 128), dtype=jnp.float32)
        for t in range(_KCH // 128):
            s = c * _KCH + t * 128
            d_t = (fsq - mm[:, s:s + 128]) + esq[:, s:s + 128]
            lt = d_t < val
            val = jnp.where(lt, d_t, val)
            tdx = jnp.where(lt, jnp.float32(t), tdx)
        mv = jnp.min(val, axis=1)
        key = tdx * 128.0 + lane
        ai = jnp.min(jnp.where(val == mv[:, None], key, jnp.float32(_K)), axis=1)
        take = mv < acc_v
        acc_i = jnp.where(take, ai + c * _KCH, acc_i)
        acc_v = jnp.where(take, mv.astype(jnp.bfloat16).astype(jnp.float32), acc_v)
    idx_ref[...] = acc_i.astype(jnp.int32).reshape(_WPG, _NCH, _CH)
    et_ref[...] = e_ref[:, pl.ds(i * _MB, _MB)].T


_dist_argmin = pl.pallas_call(
    _dist_argmin_body,
    grid=(_G,),
    in_specs=[
        pl.BlockSpec((1, _H, _W, _D), lambda i: (i, 0, 0, 0)),
        pl.BlockSpec((_D, _K), lambda i: (0, 0)),
    ],
    out_specs=[
        pl.BlockSpec((_WPG, _NCH, _CH), lambda i: (i, 0, 0)),
        pl.BlockSpec((_MB, _D), lambda i: (i, 0)),
    ],
    out_shape=[
        jax.ShapeDtypeStruct((_NW, _NCH, _CH), jnp.int32),
        jax.ShapeDtypeStruct((_K, _D), jnp.float32),
    ],
    compiler_params=pltpu.CompilerParams(dimension_semantics=("parallel",)),
)


@functools.partial(
    pl.kernel,
    mesh=plsc.VectorSubcoreMesh(core_axis_name="c", subcore_axis_name="s"),
    out_type=jax.ShapeDtypeStruct((_B, _H, _W, _D), jnp.float32),
    scratch_types=[
        pltpu.VMEM((_NCH, _CH), jnp.int32),
        pltpu.VMEM((_BPW, _D), jnp.float32),
        pltpu.SemaphoreType.DMA,
    ],
    compiler_params=pltpu.CompilerParams(use_tc_tiling_on_sc=False),
)
def _sc_gather(table_hbm, idx_hbm, out_hbm, idx_v, rows_v, sem):
    wid = lax.axis_index("s") * _NC + lax.axis_index("c")
    pltpu.sync_copy(idx_hbm.at[wid], idx_v)
    for c in range(_NCH):
        pltpu.async_copy(
            table_hbm.at[idx_v.at[c]], rows_v.at[pl.ds(c * _CH, _CH)], sem
        ).wait()
    b = wid // (_NW // _B)
    h0 = (wid % (_NW // _B)) * (_BPW // _W)
    for j in range(_BPW // _W):
        pltpu.sync_copy(
            rows_v.at[pl.ds(j * _W, _W)], out_hbm.at[b, h0 + j]
        )


def kernel(x, embed):
    idx, table = _dist_argmin(x, embed)
    return _sc_gather(table, idx)
